# Initial kernel scaffold; baseline (speedup 1.0000x reference)
import jax
import jax.numpy as jnp
from jax.experimental import pallas as pl


def kernel(x, table, W, b):
    raise NotImplementedError("write your pallas kernel here")


# speedup vs baseline: 9.5214x; 9.5214x over previous
import functools

import jax
import jax.numpy as jnp
from jax import lax
from jax.experimental import pallas as pl
from jax.experimental.pallas import tpu as pltpu
from jax.experimental.pallas import tpu_sc as plsc

_LANES = 16


def _tw_matmul(table, wp):
    V, D = table.shape
    C = wp.shape[0]
    blk = 25000 if V % 25000 == 0 else V

    def body(tbl_ref, wp_ref, out_ref):
        out_ref[...] = lax.dot_general(
            tbl_ref[...], wp_ref[...],
            dimension_numbers=(((1,), (1,)), ((), ())),
            preferred_element_type=jnp.float32)

    return pl.pallas_call(
        body,
        grid=(V // blk,),
        in_specs=[
            pl.BlockSpec((blk, D), lambda i: (i, 0)),
            pl.BlockSpec((C, D), lambda i: (0, 0)),
        ],
        out_specs=pl.BlockSpec((blk, C), lambda i: (i, 0)),
        out_shape=jax.ShapeDtypeStruct((V, C), jnp.float32),
    )(table, wp)


def _sc_pool(tw, xf, bvec, B, S):
    info = plsc.get_sparse_core_info()
    NC, NS = info.num_cores, info.num_subcores
    NW = NC * NS
    assert B % NW == 0
    BPW = B // NW
    CB = 8
    assert BPW % CB == 0
    NIT = BPW // CB
    CHUNK = CB * S

    mesh = plsc.VectorSubcoreMesh(core_axis_name="c", subcore_axis_name="s",
                                  num_cores=NC, num_subcores=NS)

    @functools.partial(
        pl.kernel,
        out_type=jax.ShapeDtypeStruct((B, _LANES), jnp.float32),
        mesh=mesh,
        compiler_params=pltpu.CompilerParams(use_tc_tiling_on_sc=False),
        scratch_types=[
            pltpu.VMEM((CHUNK,), jnp.int32),
            pltpu.VMEM((CHUNK, _LANES), jnp.float32),
            pltpu.VMEM((BPW, _LANES), jnp.float32),
            pltpu.VMEM((_LANES,), jnp.float32),
            pltpu.SemaphoreType.DMA,
        ],
    )
    def pool(tw_hbm, xf_hbm, bv_hbm, out_hbm, idx_v, rows_v, outb_v, bv_v, sem):
        wid = lax.axis_index("s") * NC + lax.axis_index("c")
        base_b = wid * BPW
        pltpu.sync_copy(bv_hbm, bv_v)
        bv = bv_v[...]

        def chunk_body(it, carry):
            off = (base_b + it * CB) * S
            pltpu.sync_copy(xf_hbm.at[pl.ds(off, CHUNK)], idx_v)
            pltpu.async_copy(tw_hbm.at[idx_v], rows_v, sem).wait()
            for bi in range(CB):
                rb = bi * S

                def grp(j, acc):
                    base = rb + j * 8
                    r0 = rows_v[base + 0]
                    r1 = rows_v[base + 1]
                    r2 = rows_v[base + 2]
                    r3 = rows_v[base + 3]
                    r4 = rows_v[base + 4]
                    r5 = rows_v[base + 5]
                    r6 = rows_v[base + 6]
                    r7 = rows_v[base + 7]
                    return acc + (((r0 + r1) + (r2 + r3))
                                  + ((r4 + r5) + (r6 + r7)))

                acc = lax.fori_loop(0, S // 8, grp,
                                    jnp.zeros((_LANES,), jnp.float32))
                outb_v[it * CB + bi] = acc * (1.0 / S) + bv
            return carry

        lax.fori_loop(0, NIT, chunk_body, 0)
        pltpu.sync_copy(outb_v, out_hbm.at[pl.ds(base_b, BPW)])

    return pool(tw, xf, bvec)


def kernel(x, table, W, b):
    B, S = x.shape
    C = W.shape[0]
    wp = jnp.zeros((_LANES, W.shape[1]), jnp.float32).at[:C].set(W)
    bvec = jnp.zeros((_LANES,), jnp.float32).at[:C].set(b)
    tw = _tw_matmul(table, wp)
    xf = x.reshape(-1).astype(jnp.int32)
    outp = _sc_pool(tw, xf, bvec, B, S)
    return outp[:, :C]


# speedup vs baseline: 13.4778x; 1.4155x over previous
import functools

import jax
import jax.numpy as jnp
from jax import lax
from jax.experimental import pallas as pl
from jax.experimental.pallas import tpu as pltpu
from jax.experimental.pallas import tpu_sc as plsc

_LANES = 16
_FOLD = 8


def _tw_matmul(table2, wp2):
    R = table2.shape[0]
    blk = 5000 if R % 5000 == 0 else R

    def body(tbl_ref, wp_ref, out_ref):
        out_ref[...] = jnp.dot(tbl_ref[...], wp_ref[...],
                               preferred_element_type=jnp.float32)

    return pl.pallas_call(
        body,
        grid=(R // blk,),
        in_specs=[
            pl.BlockSpec((blk, 256), lambda i: (i, 0)),
            pl.BlockSpec((256, 128), lambda i: (0, 0)),
        ],
        out_specs=pl.BlockSpec((blk, 128), lambda i: (i, 0)),
        out_shape=jax.ShapeDtypeStruct((R, 128), jnp.float32),
    )(table2, wp2)


def _sc_pool(tw, x, bvec):
    B, S = x.shape
    info = plsc.get_sparse_core_info()
    NC, NS = info.num_cores, info.num_subcores
    NW = NC * NS
    assert B % NW == 0
    BPW = B // NW
    CB = 8
    assert BPW % CB == 0
    NIT = BPW // CB
    assert S % 8 == 0

    mesh = plsc.VectorSubcoreMesh(core_axis_name="c", subcore_axis_name="s",
                                  num_cores=NC, num_subcores=NS)

    @functools.partial(
        pl.kernel,
        out_type=jax.ShapeDtypeStruct((B, _LANES), jnp.float32),
        mesh=mesh,
        compiler_params=pltpu.CompilerParams(use_tc_tiling_on_sc=False),
        scratch_types=[
            pltpu.VMEM((CB, S), jnp.int32),
            pltpu.VMEM((CB * S, _LANES), jnp.float32),
            pltpu.VMEM((BPW, _LANES), jnp.float32),
            pltpu.VMEM((_LANES,), jnp.float32),
            pltpu.SemaphoreType.DMA,
        ],
    )
    def pool(tw_hbm, x_hbm, bv_hbm, out_hbm, idx_v, rows_v, outb_v, bv_v, sem):
        wid = lax.axis_index("s") * NC + lax.axis_index("c")
        base_b = wid * BPW
        pltpu.sync_copy(bv_hbm, bv_v)
        bv = bv_v[...]

        def chunk_body(it, carry):
            row0 = base_b + it * CB
            pltpu.sync_copy(x_hbm.at[pl.ds(row0, CB)], idx_v)
            cps = [
                pltpu.async_copy(tw_hbm.at[idx_v.at[bi]],
                                 rows_v.at[pl.ds(bi * S, S)], sem)
                for bi in range(CB)
            ]
            for cp in cps:
                cp.wait()
            for bi in range(CB):
                rb = bi * S

                def grp(j, acc):
                    base = rb + j * 8
                    r0 = rows_v[base + 0]
                    r1 = rows_v[base + 1]
                    r2 = rows_v[base + 2]
                    r3 = rows_v[base + 3]
                    r4 = rows_v[base + 4]
                    r5 = rows_v[base + 5]
                    r6 = rows_v[base + 6]
                    r7 = rows_v[base + 7]
                    return acc + (((r0 + r1) + (r2 + r3))
                                  + ((r4 + r5) + (r6 + r7)))

                acc = lax.fori_loop(0, S // 8, grp,
                                    jnp.zeros((_LANES,), jnp.float32))
                outb_v[it * CB + bi] = acc * (1.0 / S) + bv
            return carry

        lax.fori_loop(0, NIT, chunk_body, 0)
        pltpu.sync_copy(outb_v, out_hbm.at[pl.ds(base_b, BPW)])

    return pool(tw, x, bvec)


def kernel(x, table, W, b):
    B, S = x.shape
    V, D = table.shape
    C = W.shape[0]
    assert V % _FOLD == 0
    wp = jnp.zeros((_LANES, D), jnp.float32).at[:C].set(W)
    eye8 = jnp.eye(_FOLD, dtype=jnp.float32)
    wp2 = jnp.einsum("st,cd->sdtc", eye8, wp).reshape(_FOLD * D,
                                                      _FOLD * _LANES)
    bvec = jnp.zeros((_LANES,), jnp.float32).at[:C].set(b)
    table2 = table.reshape(V // _FOLD, _FOLD * D)
    tw = _tw_matmul(table2, wp2).reshape(V, _LANES)
    outp = _sc_pool(tw, x.astype(jnp.int32), bvec)
    return outp[:, :C]
